# baseline (speedup 1.0000x reference)
import functools

import jax
import jax.numpy as jnp
from jax import lax
from jax.experimental import pallas as pl
from jax.experimental.pallas import tpu as pltpu
from jax.experimental.pallas import tpu_sc as plsc

_D = 64
_S = 200
_B = 4096
_EPS = 1e-6
_N = _B * _S
_NW = 32
_PER_W = _N // _NW
_CHUNK = _S
_NCH = _PER_W // _CHUNK
_HALF = _CHUNK // 2
_NBUF = 2
_L = 16


def _rsqrt(a):
    i = lax.bitcast_convert_type(a, jnp.int32)
    i = jnp.int32(0x5F3759DF) - lax.shift_right_logical(i, 1)
    y = lax.bitcast_convert_type(i, jnp.float32)
    ah = a * jnp.float32(0.5)
    y = y * (jnp.float32(1.5) - ah * y * y)
    y = y * (jnp.float32(1.5) - ah * y * y)
    y = y * (jnp.float32(1.5) - ah * y * y)
    return y


_mesh = plsc.VectorSubcoreMesh(core_axis_name="c", subcore_axis_name="s")


@functools.partial(
    pl.kernel,
    out_type=jax.ShapeDtypeStruct((_N, _D), jnp.float32),
    mesh=_mesh,
    scratch_types=[
        pltpu.VMEM((2 * _NCH, _HALF), jnp.int32),
        pltpu.VMEM((_S, _D), jnp.float32),
        pltpu.VMEM((2, _D), jnp.float32),
        pltpu.VMEM((_CHUNK, _D), jnp.float32),
        pltpu.VMEM((_CHUNK, _D), jnp.float32),
        pltpu.VMEM((_CHUNK, _D), jnp.float32),
        pltpu.VMEM((_CHUNK, _D), jnp.float32),
        pltpu.SemaphoreType.DMA,
        pltpu.SemaphoreType.DMA,
        pltpu.SemaphoreType.DMA,
        pltpu.SemaphoreType.DMA,
    ],
    compiler_params=pltpu.CompilerParams(
        needs_layout_passes=False, use_tc_tiling_on_sc=False
    ),
)
def _emb(ids_hbm, tok_hbm, pos_hbm, scale_hbm, bias_hbm, out_hbm,
         ids_v, pos_v, sb_v, in0, in1, ot0, ot1, sg0, sg1, so0, so1):
    wid = lax.axis_index("s") * 2 + lax.axis_index("c")
    in_bufs = (in0, in1)
    out_bufs = (ot0, ot1)
    sem_g = (sg0, sg1)
    sem_o = (so0, so1)

    pltpu.sync_copy(ids_hbm.at[wid], ids_v)
    pltpu.sync_copy(pos_hbm.at[pl.ds(0, _S)], pos_v)
    pltpu.sync_copy(scale_hbm, sb_v.at[0])
    pltpu.sync_copy(bias_hbm, sb_v.at[1])

    scale_r = [sb_v[0, pl.ds(_L * j, _L)] for j in range(_D // _L)]
    bias_r = [sb_v[1, pl.ds(_L * j, _L)] for j in range(_D // _L)]

    def start_gather(g, buf, sem):
        pltpu.async_copy(tok_hbm.at[ids_v.at[2 * g]],
                         buf.at[pl.ds(0, _HALF)], sem)
        pltpu.async_copy(tok_hbm.at[ids_v.at[2 * g + 1]],
                         buf.at[pl.ds(_HALF, _HALF)], sem)

    def wait_bytes(dst, sem):
        pltpu.make_async_copy(tok_hbm.at[pl.ds(0, dst.shape[0])], dst, sem).wait()

    def compute(buf_in, buf_out):
        @plsc.parallel_loop(0, _CHUNK, unroll=4)
        def _row(i):
            x = [buf_in[i, pl.ds(_L * j, _L)] + pos_v[i, pl.ds(_L * j, _L)]
                 for j in range(_D // _L)]
            ssum = (x[0] + x[1]) + (x[2] + x[3])
            mean = jnp.broadcast_to(jnp.sum(ssum) * jnp.float32(1.0 / _D), (_L,))
            d = [xj - mean for xj in x]
            vsum = (d[0] * d[0] + d[1] * d[1]) + (d[2] * d[2] + d[3] * d[3])
            var = jnp.broadcast_to(jnp.sum(vsum) * jnp.float32(1.0 / _D), (_L,))
            r = _rsqrt(var + jnp.float32(_EPS))
            for j in range(_D // _L):
                buf_out[i, pl.ds(_L * j, _L)] = d[j] * (r * scale_r[j]) + bias_r[j]

    for b in range(_NBUF):
        start_gather(b, in_bufs[b], sem_g[b])

    def outer(m, carry):
        base = m * _NBUF
        for b in range(_NBUF):
            g = base + b
            wait_bytes(in_bufs[b], sem_g[b])

            @pl.when(g >= _NBUF)
            def _():
                pltpu.make_async_copy(out_bufs[b], out_hbm.at[pl.ds(0, _CHUNK)],
                                      sem_o[b]).wait()

            compute(in_bufs[b], out_bufs[b])
            row0 = (wid * _NCH + g) * _CHUNK
            pltpu.async_copy(out_bufs[b], out_hbm.at[pl.ds(row0, _CHUNK)],
                             sem_o[b])

            @pl.when(g + _NBUF < _NCH)
            def _():
                start_gather(g + _NBUF, in_bufs[b], sem_g[b])

        return carry

    lax.fori_loop(0, _NCH // _NBUF, outer, jnp.int32(0))

    for b in range(_NBUF):
        pltpu.make_async_copy(out_bufs[b], out_hbm.at[pl.ds(0, _CHUNK)],
                              sem_o[b]).wait()


def kernel(input_ids, token_emb_w, pos_emb_w, ln_scale, ln_bias):
    ids = input_ids.reshape(_NW, 2 * _NCH, _HALF)
    out = _emb(ids, token_emb_w, pos_emb_w, ln_scale, ln_bias)
    return out.reshape(_B, _S, _D)
